# baseline (speedup 1.0000x reference)
import functools

import jax
import jax.numpy as jnp
from jax import lax
from jax.experimental import pallas as pl
from jax.experimental.pallas import tpu as pltpu
from jax.experimental.pallas import tpu_sc as plsc

VOCAB = 1000
EMBED_DIM = 4000
PAD_DIM = 4096
BATCH = 4096
P = 4
BP = BATCH // P

_info = plsc.get_sparse_core_info()
_NC, _NS = _info.num_cores, _info.num_subcores
NW = _NC * _NS
BPW = BP // NW
CH = 8
NCHUNK = BPW // CH
NBUF = 3


def _body(idx_hbm, table_hbm, out_hbm, idx_v, *bufs_and_sems):
    bufs = bufs_and_sems[:NBUF]
    gsems = bufs_and_sems[NBUF:2 * NBUF]
    osems = bufs_and_sems[2 * NBUF:3 * NBUF]

    wid = lax.axis_index("s") * _NC + lax.axis_index("c")
    base = wid * BPW

    pltpu.sync_copy(idx_hbm.at[pl.ds(base, BPW)], idx_v)

    def gather(c, s):
        return pltpu.async_copy(
            table_hbm.at[idx_v.at[pl.ds(c * CH, CH)]], bufs[s], gsems[s])

    nbuf = min(NBUF, NCHUNK)
    gc = [gather(s, s) for s in range(nbuf)]
    oc = [None] * nbuf
    for c in range(NCHUNK):
        s = c % nbuf
        gc[s].wait()
        oc[s] = pltpu.async_copy(
            bufs[s], out_hbm.at[pl.ds(base + c * CH, CH)], osems[s])
        nxt = c + nbuf
        if nxt < NCHUNK:
            oc[s].wait()
            gc[s] = gather(nxt, s)
    for s in range(nbuf):
        oc[s].wait()


def _make_call():
    mesh = plsc.VectorSubcoreMesh(core_axis_name="c", subcore_axis_name="s")
    return functools.partial(
        pl.kernel,
        mesh=mesh,
        out_type=jax.ShapeDtypeStruct((BP, PAD_DIM), jnp.float32),
        scratch_types=(
            [pltpu.VMEM((BPW,), jnp.int32)]
            + [pltpu.VMEM((CH, PAD_DIM), jnp.float32)] * NBUF
            + [pltpu.SemaphoreType.DMA] * (2 * NBUF)
        ),
    )(_body)


_gather_call = _make_call()

_RP_ROWS = 256


def _seed_body(part_ref, out_ref):
    out_ref[...] = part_ref[:, :EMBED_DIM]


def _write_body(out_in, part_ref, out_ref):
    del out_in
    out_ref[...] = part_ref[:, :EMBED_DIM]


def _repack(out, part, p):
    grid = (BP // _RP_ROWS,)
    part_spec = pl.BlockSpec((_RP_ROWS, PAD_DIM), lambda i: (i, 0))
    out_spec = pl.BlockSpec(
        (_RP_ROWS, EMBED_DIM), lambda i, _p=p: (_p * BP // _RP_ROWS + i, 0))
    out_shape = jax.ShapeDtypeStruct((BATCH, EMBED_DIM), jnp.float32)
    if out is None:
        return pl.pallas_call(
            _seed_body, grid=grid, in_specs=[part_spec],
            out_specs=out_spec, out_shape=out_shape)(part)
    return pl.pallas_call(
        _write_body, grid=grid,
        in_specs=[pl.BlockSpec(memory_space=pl.ANY), part_spec],
        out_specs=out_spec, out_shape=out_shape,
        input_output_aliases={0: 0})(out, part)


def kernel(tokens, embedding_table):
    idx = tokens.astype(jnp.int32)
    table_p = jnp.pad(embedding_table, ((0, 0), (0, PAD_DIM - EMBED_DIM)))
    out = None
    for p in range(P):
        part = _gather_call(lax.slice(idx, (p * BP,), ((p + 1) * BP,)),
                            table_p)
        out = _repack(out, part, p)
    return out


# speedup vs baseline: 1.0248x; 1.0248x over previous
import functools

import jax
import jax.numpy as jnp
from jax import lax
from jax.experimental import pallas as pl
from jax.experimental.pallas import tpu as pltpu
from jax.experimental.pallas import tpu_sc as plsc

VOCAB = 1000
EMBED_DIM = 4000
PAD_DIM = 4096
BATCH = 4096
P = 2
BP = BATCH // P

_info = plsc.get_sparse_core_info()
_NC, _NS = _info.num_cores, _info.num_subcores
NW = _NC * _NS
BPW = BP // NW
CH = 8
NCHUNK = BPW // CH
NBUF = 3


def _body(idx_hbm, table_hbm, out_hbm, idx_v, *bufs_and_sems):
    bufs = bufs_and_sems[:NBUF]
    gsems = bufs_and_sems[NBUF:2 * NBUF]
    osems = bufs_and_sems[2 * NBUF:3 * NBUF]

    wid = lax.axis_index("s") * _NC + lax.axis_index("c")
    base = wid * BPW

    pltpu.sync_copy(idx_hbm.at[pl.ds(base, BPW)], idx_v)

    def gather(c, s):
        return pltpu.async_copy(
            table_hbm.at[idx_v.at[pl.ds(c * CH, CH)]], bufs[s], gsems[s])

    nbuf = min(NBUF, NCHUNK)
    gc = [gather(s, s) for s in range(nbuf)]
    oc = [None] * nbuf
    for c in range(NCHUNK):
        s = c % nbuf
        gc[s].wait()
        oc[s] = pltpu.async_copy(
            bufs[s], out_hbm.at[pl.ds(base + c * CH, CH)], osems[s])
        nxt = c + nbuf
        if nxt < NCHUNK:
            oc[s].wait()
            gc[s] = gather(nxt, s)
    for s in range(nbuf):
        oc[s].wait()


def _make_call():
    mesh = plsc.VectorSubcoreMesh(core_axis_name="c", subcore_axis_name="s")
    return functools.partial(
        pl.kernel,
        mesh=mesh,
        out_type=jax.ShapeDtypeStruct((BP, PAD_DIM), jnp.float32),
        scratch_types=(
            [pltpu.VMEM((BPW,), jnp.int32)]
            + [pltpu.VMEM((CH, PAD_DIM), jnp.float32)] * NBUF
            + [pltpu.SemaphoreType.DMA] * (2 * NBUF)
        ),
    )(_body)


_gather_call = _make_call()

_RP_ROWS = 256


def _seed_body(part_ref, out_ref):
    out_ref[...] = part_ref[:, :EMBED_DIM]


def _write_body(out_in, part_ref, out_ref):
    del out_in
    out_ref[...] = part_ref[:, :EMBED_DIM]


def _repack(out, part, p):
    grid = (BP // _RP_ROWS,)
    part_spec = pl.BlockSpec((_RP_ROWS, PAD_DIM), lambda i: (i, 0))
    out_spec = pl.BlockSpec(
        (_RP_ROWS, EMBED_DIM), lambda i, _p=p: (_p * BP // _RP_ROWS + i, 0))
    out_shape = jax.ShapeDtypeStruct((BATCH, EMBED_DIM), jnp.float32)
    if out is None:
        return pl.pallas_call(
            _seed_body, grid=grid, in_specs=[part_spec],
            out_specs=out_spec, out_shape=out_shape)(part)
    return pl.pallas_call(
        _write_body, grid=grid,
        in_specs=[pl.BlockSpec(memory_space=pl.ANY), part_spec],
        out_specs=out_spec, out_shape=out_shape,
        input_output_aliases={0: 0})(out, part)


def kernel(tokens, embedding_table):
    idx = tokens.astype(jnp.int32)
    table_p = jnp.pad(embedding_table, ((0, 0), (0, PAD_DIM - EMBED_DIM)))
    part0 = _gather_call(lax.slice(idx, (0,), (BP,)), table_p)
    out0 = jnp.pad(part0[:, :EMBED_DIM], ((0, BATCH - BP), (0, 0)))
    part1 = _gather_call(lax.slice(idx, (BP,), (BATCH,)), table_p)
    return lax.dynamic_update_slice(out0, part1[:, :EMBED_DIM], (BP, 0))


# speedup vs baseline: 1.5948x; 1.5562x over previous
import functools

import jax
import jax.numpy as jnp
from jax import lax
from jax.experimental import pallas as pl
from jax.experimental.pallas import tpu as pltpu
from jax.experimental.pallas import tpu_sc as plsc

VOCAB = 1000
EMBED_DIM = 4000
PAD_DIM = 4096
MAIN = 3968
BATCH = 4096

_info = plsc.get_sparse_core_info()
_NC, _NS = _info.num_cores, _info.num_subcores
NW = _NC * _NS
BPW = BATCH // NW
CH = 8
NCHUNK = BPW // CH
NBUF = 3


def _body(idx_hbm, table_hbm, tail_tbl_hbm, out_hbm, idx_v, *refs):
    mbufs = refs[:NBUF]
    tbufs = refs[NBUF:2 * NBUF]
    gsems = refs[2 * NBUF:3 * NBUF]
    hsems = refs[3 * NBUF:4 * NBUF]
    osems = refs[4 * NBUF:5 * NBUF]
    tsems = refs[5 * NBUF:6 * NBUF]

    wid = lax.axis_index("s") * _NC + lax.axis_index("c")
    base = wid * BPW

    pltpu.sync_copy(idx_hbm.at[pl.ds(base, BPW)], idx_v)

    def gather(c, s):
        ids = idx_v.at[pl.ds(c * CH, CH)]
        return (
            pltpu.async_copy(
                table_hbm.at[ids, pl.ds(0, MAIN)], mbufs[s], gsems[s]),
            pltpu.async_copy(tail_tbl_hbm.at[ids], tbufs[s], hsems[s]),
        )

    gc = [gather(s, s) for s in range(NBUF)]
    oc = [None] * NBUF
    for c in range(NCHUNK):
        s = c % NBUF
        rows = pl.ds(base + c * CH, CH)
        gc[s][0].wait()
        gc[s][1].wait()
        oc[s] = (
            pltpu.async_copy(
                mbufs[s], out_hbm.at[rows, pl.ds(0, MAIN)], osems[s]),
            pltpu.async_copy(
                tbufs[s], out_hbm.at[rows, pl.ds(MAIN, 128)], tsems[s]),
        )
        nxt = c + NBUF
        if nxt < NCHUNK:
            oc[s][0].wait()
            oc[s][1].wait()
            gc[s] = gather(nxt, s)
    for s in range(NBUF):
        oc[s][0].wait()
        oc[s][1].wait()


def _make_call():
    mesh = plsc.VectorSubcoreMesh(core_axis_name="c", subcore_axis_name="s")
    return functools.partial(
        pl.kernel,
        mesh=mesh,
        out_type=jax.ShapeDtypeStruct((BATCH, PAD_DIM), jnp.float32),
        scratch_types=(
            [pltpu.VMEM((BPW,), jnp.int32)]
            + [pltpu.VMEM((CH, MAIN), jnp.float32)] * NBUF
            + [pltpu.VMEM((CH, 128), jnp.float32)] * NBUF
            + [pltpu.SemaphoreType.DMA] * (4 * NBUF)
        ),
    )(_body)


_gather_call = _make_call()


def kernel(tokens, embedding_table):
    idx = tokens.astype(jnp.int32)
    tail_tbl = jnp.pad(
        lax.slice(embedding_table, (0, MAIN), (VOCAB, EMBED_DIM)),
        ((0, 0), (0, 128 - (EMBED_DIM - MAIN))))
    out_p = _gather_call(idx, embedding_table, tail_tbl)
    return out_p[:, :EMBED_DIM]


# speedup vs baseline: 1.5968x; 1.0013x over previous
import functools

import jax
import jax.numpy as jnp
from jax import lax
from jax.experimental import pallas as pl
from jax.experimental.pallas import tpu as pltpu
from jax.experimental.pallas import tpu_sc as plsc

VOCAB = 1000
EMBED_DIM = 4000
PAD_DIM = 4096
MAIN = 3968
BATCH = 4096

_info = plsc.get_sparse_core_info()
_NC, _NS = _info.num_cores, _info.num_subcores
NW = _NC * _NS
BPW = BATCH // NW
CH = 8
NCHUNK = BPW // CH
NBUF = 4
TBUF = 2


def _body(idx_hbm, table_hbm, tail_tbl_hbm, out_hbm, idx_v, *refs):
    mbufs = refs[:NBUF]
    tbufs = refs[NBUF:NBUF + TBUF]
    r = NBUF + TBUF
    gsems = refs[r:r + NBUF]
    hsems = refs[r + NBUF:r + NBUF + TBUF]
    osems = refs[r + NBUF + TBUF:r + 2 * NBUF + TBUF]
    tsems = refs[r + 2 * NBUF + TBUF:r + 2 * NBUF + 2 * TBUF]

    wid = lax.axis_index("s") * _NC + lax.axis_index("c")
    base = wid * BPW

    pltpu.sync_copy(idx_hbm.at[pl.ds(base, BPW)], idx_v)

    def gather_main(c, s):
        ids = idx_v.at[pl.ds(c * CH, CH)]
        return pltpu.async_copy(
            table_hbm.at[ids, pl.ds(0, MAIN)], mbufs[s], gsems[s])

    def gather_tail(c, t):
        ids = idx_v.at[pl.ds(c * CH, CH)]
        return pltpu.async_copy(tail_tbl_hbm.at[ids], tbufs[t], hsems[t])

    gm = [gather_main(s, s) for s in range(NBUF)]
    gt = [gather_tail(t, t) for t in range(TBUF)]
    om = [None] * NBUF
    ot = [None] * TBUF
    for c in range(NCHUNK):
        s, t = c % NBUF, c % TBUF
        rows = pl.ds(base + c * CH, CH)
        gm[s].wait()
        gt[t].wait()
        om[s] = pltpu.async_copy(
            mbufs[s], out_hbm.at[rows, pl.ds(0, MAIN)], osems[s])
        ot[t] = pltpu.async_copy(
            tbufs[t], out_hbm.at[rows, pl.ds(MAIN, 128)], tsems[t])
        if c + TBUF < NCHUNK:
            ot[t].wait()
            gt[t] = gather_tail(c + TBUF, t)
        if c + NBUF < NCHUNK:
            om[s].wait()
            gm[s] = gather_main(c + NBUF, s)
    for s in range(NBUF):
        om[s].wait()
    for t in range(TBUF):
        ot[t].wait()


def _make_call():
    mesh = plsc.VectorSubcoreMesh(core_axis_name="c", subcore_axis_name="s")
    return functools.partial(
        pl.kernel,
        mesh=mesh,
        out_type=jax.ShapeDtypeStruct((BATCH, PAD_DIM), jnp.float32),
        scratch_types=(
            [pltpu.VMEM((BPW,), jnp.int32)]
            + [pltpu.VMEM((CH, MAIN), jnp.float32)] * NBUF
            + [pltpu.VMEM((CH, 128), jnp.float32)] * TBUF
            + [pltpu.SemaphoreType.DMA] * (2 * NBUF + 2 * TBUF)
        ),
    )(_body)


_gather_call = _make_call()


def kernel(tokens, embedding_table):
    idx = tokens.astype(jnp.int32)
    tail_tbl = jnp.pad(
        lax.slice(embedding_table, (0, MAIN), (VOCAB, EMBED_DIM)),
        ((0, 0), (0, 128 - (EMBED_DIM - MAIN))))
    out_p = _gather_call(idx, embedding_table, tail_tbl)
    return out_p[:, :EMBED_DIM]


# speedup vs baseline: 1.5987x; 1.0012x over previous
import functools

import jax
import jax.numpy as jnp
from jax import lax
from jax.experimental import pallas as pl
from jax.experimental.pallas import tpu as pltpu
from jax.experimental.pallas import tpu_sc as plsc

VOCAB = 1000
EMBED_DIM = 4000
PAD_DIM = 4096
MAIN = 3968
BATCH = 4096

_info = plsc.get_sparse_core_info()
_NC, _NS = _info.num_cores, _info.num_subcores
NW = _NC * _NS
BPW = BATCH // NW
CH = 8
NCHUNK = BPW // CH
NBUF = 4
TBUF = 2


def _body(idx_hbm, table_hbm, tail_tbl_hbm, out_hbm, idx_v, *refs):
    mbufs = refs[:NBUF]
    tbufs = refs[NBUF:NBUF + TBUF]
    r = NBUF + TBUF
    gsems = refs[r:r + NBUF]
    hsems = refs[r + NBUF:r + NBUF + TBUF]
    osems = refs[r + NBUF + TBUF:r + 2 * NBUF + TBUF]
    tsems = refs[r + 2 * NBUF + TBUF:r + 2 * NBUF + 2 * TBUF]

    wid = lax.axis_index("s") * _NC + lax.axis_index("c")
    base = wid * BPW

    pltpu.sync_copy(idx_hbm.at[pl.ds(base, BPW)], idx_v)

    def gather_main(c, s):
        ids = idx_v.at[pl.ds(c * CH, CH)]
        return pltpu.async_copy(
            table_hbm.at[ids, pl.ds(0, MAIN)], mbufs[s], gsems[s])

    def gather_tail(c, t):
        ids = idx_v.at[pl.ds(c * CH, CH)]
        return pltpu.async_copy(tail_tbl_hbm.at[ids], tbufs[t], hsems[t])

    gm = [gather_main(s, s) for s in range(NBUF)]
    gt = [gather_tail(t, t) for t in range(TBUF)]
    om = [None] * NBUF
    ot = [None] * TBUF
    for c in range(NCHUNK):
        s, t = c % NBUF, c % TBUF
        rows = pl.ds(base + c * CH, CH)
        gm[s].wait()
        gt[t].wait()
        om[s] = pltpu.async_copy(
            mbufs[s], out_hbm.at[rows, pl.ds(0, MAIN)], osems[s])
        ot[t] = pltpu.async_copy(
            tbufs[t], out_hbm.at[rows, pl.ds(MAIN, 128)], tsems[t])
        if c + TBUF < NCHUNK:
            ot[t].wait()
            gt[t] = gather_tail(c + TBUF, t)
        if c + NBUF < NCHUNK:
            om[s].wait()
            gm[s] = gather_main(c + NBUF, s)
    for s in range(NBUF):
        om[s].wait()
    for t in range(TBUF):
        ot[t].wait()


def _make_call():
    mesh = plsc.VectorSubcoreMesh(core_axis_name="c", subcore_axis_name="s")
    return functools.partial(
        pl.kernel,
        mesh=mesh,
        out_type=jax.ShapeDtypeStruct((BATCH, PAD_DIM), jnp.float32),
        scratch_types=(
            [pltpu.VMEM((BPW,), jnp.int32)]
            + [pltpu.VMEM((CH, MAIN), jnp.float32)] * NBUF
            + [pltpu.VMEM((CH, 128), jnp.float32)] * TBUF
            + [pltpu.SemaphoreType.DMA] * (2 * NBUF + 2 * TBUF)
        ),
    )(_body)


_gather_call = _make_call()


def kernel(tokens, embedding_table):
    idx = tokens.astype(jnp.int32)
    tail_tbl = jnp.pad(
        lax.slice(embedding_table, (0, MAIN), (VOCAB, EMBED_DIM)),
        ((0, 0), (0, 128 - (EMBED_DIM - MAIN))))
    out_p = _gather_call(idx, embedding_table, tail_tbl)
    return out_p[:, :EMBED_DIM] * jnp.float32(1.0)


# speedup vs baseline: 1.6032x; 1.0028x over previous
import functools

import jax
import jax.numpy as jnp
from jax import lax
from jax.experimental import pallas as pl
from jax.experimental.pallas import tpu as pltpu
from jax.experimental.pallas import tpu_sc as plsc

VOCAB = 1000
EMBED_DIM = 4000
PAD_DIM = 4096
MAIN = 3968
BATCH = 4096

_info = plsc.get_sparse_core_info()
_NC, _NS = _info.num_cores, _info.num_subcores
NW = _NC * _NS
BPW = BATCH // NW
CH = 16
NCHUNK = BPW // CH
NBUF = 2
TBUF = 1


def _body(idx_hbm, table_hbm, tail_tbl_hbm, out_hbm, idx_v, *refs):
    mbufs = refs[:NBUF]
    tbufs = refs[NBUF:NBUF + TBUF]
    r = NBUF + TBUF
    gsems = refs[r:r + NBUF]
    hsems = refs[r + NBUF:r + NBUF + TBUF]
    osems = refs[r + NBUF + TBUF:r + 2 * NBUF + TBUF]
    tsems = refs[r + 2 * NBUF + TBUF:r + 2 * NBUF + 2 * TBUF]

    wid = lax.axis_index("s") * _NC + lax.axis_index("c")
    base = wid * BPW

    pltpu.sync_copy(idx_hbm.at[pl.ds(base, BPW)], idx_v)

    def gather_main(c, s):
        ids = idx_v.at[pl.ds(c * CH, CH)]
        return pltpu.async_copy(
            table_hbm.at[ids, pl.ds(0, MAIN)], mbufs[s], gsems[s])

    def gather_tail(c, t):
        ids = idx_v.at[pl.ds(c * CH, CH)]
        return pltpu.async_copy(tail_tbl_hbm.at[ids], tbufs[t], hsems[t])

    gm = [gather_main(s, s) for s in range(NBUF)]
    gt = [gather_tail(t, t) for t in range(TBUF)]
    om = [None] * NBUF
    ot = [None] * TBUF
    for c in range(NCHUNK):
        s, t = c % NBUF, c % TBUF
        rows = pl.ds(base + c * CH, CH)
        gm[s].wait()
        gt[t].wait()
        om[s] = pltpu.async_copy(
            mbufs[s], out_hbm.at[rows, pl.ds(0, MAIN)], osems[s])
        ot[t] = pltpu.async_copy(
            tbufs[t], out_hbm.at[rows, pl.ds(MAIN, 128)], tsems[t])
        if c + TBUF < NCHUNK:
            ot[t].wait()
            gt[t] = gather_tail(c + TBUF, t)
        if c + NBUF < NCHUNK:
            om[s].wait()
            gm[s] = gather_main(c + NBUF, s)
    for s in range(NBUF):
        om[s].wait()
    for t in range(TBUF):
        ot[t].wait()


def _make_call():
    mesh = plsc.VectorSubcoreMesh(core_axis_name="c", subcore_axis_name="s")
    return functools.partial(
        pl.kernel,
        mesh=mesh,
        out_type=jax.ShapeDtypeStruct((BATCH, PAD_DIM), jnp.float32),
        scratch_types=(
            [pltpu.VMEM((BPW,), jnp.int32)]
            + [pltpu.VMEM((CH, MAIN), jnp.float32)] * NBUF
            + [pltpu.VMEM((CH, 128), jnp.float32)] * TBUF
            + [pltpu.SemaphoreType.DMA] * (2 * NBUF + 2 * TBUF)
        ),
    )(_body)


_gather_call = _make_call()


def kernel(tokens, embedding_table):
    idx = tokens.astype(jnp.int32)
    tail_tbl = jnp.pad(
        lax.slice(embedding_table, (0, MAIN), (VOCAB, EMBED_DIM)),
        ((0, 0), (0, 128 - (EMBED_DIM - MAIN))))
    out_p = _gather_call(idx, embedding_table, tail_tbl)
    return out_p[:, :EMBED_DIM]
